# baseline (speedup 1.0000x reference)
import functools

import jax
import jax.numpy as jnp
from jax import lax
from jax.experimental import pallas as pl
from jax.experimental.pallas import tpu as pltpu
from jax.experimental.pallas import tpu_sc as plsc

B = 16384
EMB = 32
NW = 32
BPW = B // NW
CHUNK = 128
NCHUNK = BPW // CHUNK


def _sc_kernel(user_ids_h, pos_ids_h, neg_ids_h, user_emb_h, movie_emb_h,
               out_pos_h, out_neg_h,
               uidx_v, pidx_v, nidx_v, u_v, p_v, n_v, op_v, on_v, sem):
    wid = lax.axis_index("s") * 2 + lax.axis_index("c")

    pltpu.sync_copy(user_ids_h.at[wid], uidx_v)
    pltpu.sync_copy(pos_ids_h.at[wid], pidx_v)
    pltpu.sync_copy(neg_ids_h.at[wid], nidx_v)

    copies = []
    for j in range(NCHUNK):
        dst = pl.ds(j * CHUNK, CHUNK)
        copies.append(pltpu.async_copy(user_emb_h.at[uidx_v.at[j]], u_v.at[dst], sem))
        copies.append(pltpu.async_copy(movie_emb_h.at[pidx_v.at[j]], p_v.at[dst], sem))
        copies.append(pltpu.async_copy(movie_emb_h.at[nidx_v.at[j]], n_v.at[dst], sem))
    for c in copies:
        c.wait()

    def block(blk, carry):
        base = blk * 16
        rows = base + lax.iota(jnp.int32, 16)
        accp = jnp.zeros((16,), jnp.float32)
        accn = jnp.zeros((16,), jnp.float32)
        for d in range(EMB):
            col = jnp.full((16,), d, jnp.int32)
            ug = plsc.load_gather(u_v, [rows, col])
            pg = plsc.load_gather(p_v, [rows, col])
            ng = plsc.load_gather(n_v, [rows, col])
            accp = accp + ug * pg
            accn = accn + ug * ng
        op_v[pl.ds(base, 16)] = accp
        on_v[pl.ds(base, 16)] = accn
        return carry

    lax.fori_loop(0, BPW // 16, block, 0)

    out = pl.ds(wid * BPW, BPW)
    pltpu.sync_copy(op_v, out_pos_h.at[out])
    pltpu.sync_copy(on_v, out_neg_h.at[out])


@jax.jit
def kernel(user_ids, pos_ids, neg_ids, user_emb, movie_emb):
    uids = user_ids.astype(jnp.int32).reshape(NW, NCHUNK, CHUNK)
    pids = pos_ids.astype(jnp.int32).reshape(NW, NCHUNK, CHUNK)
    nids = neg_ids.astype(jnp.int32).reshape(NW, NCHUNK, CHUNK)

    mesh = plsc.VectorSubcoreMesh(
        core_axis_name="c", subcore_axis_name="s", num_cores=2, num_subcores=16
    )
    run = pl.kernel(
        _sc_kernel,
        out_type=(
            jax.ShapeDtypeStruct((B,), jnp.float32),
            jax.ShapeDtypeStruct((B,), jnp.float32),
        ),
        mesh=mesh,
        scratch_types=[
            pltpu.VMEM((NCHUNK, CHUNK), jnp.int32),
            pltpu.VMEM((NCHUNK, CHUNK), jnp.int32),
            pltpu.VMEM((NCHUNK, CHUNK), jnp.int32),
            pltpu.VMEM((BPW, EMB), jnp.float32),
            pltpu.VMEM((BPW, EMB), jnp.float32),
            pltpu.VMEM((BPW, EMB), jnp.float32),
            pltpu.VMEM((BPW,), jnp.float32),
            pltpu.VMEM((BPW,), jnp.float32),
            pltpu.SemaphoreType.DMA,
        ],
        compiler_params=pltpu.CompilerParams(
            needs_layout_passes=False, use_tc_tiling_on_sc=False
        ),
    )
    return run(uids, pids, nids, user_emb, movie_emb)


# speedup vs baseline: 1.4780x; 1.4780x over previous
import jax
import jax.numpy as jnp
from jax import lax
from jax.experimental import pallas as pl
from jax.experimental.pallas import tpu as pltpu
from jax.experimental.pallas import tpu_sc as plsc

B = 16384
EMB = 32
NW = 32
BPW = B // NW
CHUNK = 128
NCHUNK = BPW // CHUNK


def _sc_kernel(user_ids_h, pos_ids_h, neg_ids_h, user_emb_h, movie_emb_h,
               out_pos_h, out_neg_h,
               uidx_v, pidx_v, nidx_v,
               u_b0, u_b1, p_b0, p_b1, n_b0, n_b1,
               op_v, on_v, sem0, sem1):
    wid = lax.axis_index("s") * 2 + lax.axis_index("c")

    pltpu.sync_copy(user_ids_h.at[pl.ds(wid * BPW, BPW)], uidx_v)
    pltpu.sync_copy(pos_ids_h.at[pl.ds(wid * BPW, BPW)], pidx_v)
    pltpu.sync_copy(neg_ids_h.at[pl.ds(wid * BPW, BPW)], nidx_v)

    bufs = ((u_b0, p_b0, n_b0), (u_b1, p_b1, n_b1))
    sems = (sem0, sem1)

    def fire_chunk(j):
        u_b, p_b, n_b = bufs[j % 2]
        sem = sems[j % 2]

        def grp(g, carry):
            src = pl.ds(j * CHUNK + g * 16, 16)
            uids16 = uidx_v[src]
            pids16 = pidx_v[src]
            nids16 = nidx_v[src]
            for k in range(16):
                dst = g * 16 + k
                pltpu.async_copy(
                    user_emb_h.at[pl.ds(uids16[k], 1)], u_b.at[dst], sem)
                pltpu.async_copy(
                    movie_emb_h.at[pl.ds(pids16[k], 1)], p_b.at[dst], sem)
                pltpu.async_copy(
                    movie_emb_h.at[pl.ds(nids16[k], 1)], n_b.at[dst], sem)
            return carry

        lax.fori_loop(0, CHUNK // 16, grp, 0)

    def wait_chunk(j):
        sem = sems[j % 2]
        for buf in bufs[j % 2]:
            pltpu.make_async_copy(
                user_emb_h.at[pl.ds(0, CHUNK)],
                buf.at[pl.ds(0, CHUNK), 0],
                sem,
            ).wait()

    def compute_chunk(j):
        u_b, p_b, n_b = bufs[j % 2]

        def block(blk, carry):
            rows = blk * 16 + lax.iota(jnp.int32, 16)
            zero = jnp.zeros((16,), jnp.int32)
            accp = jnp.zeros((16,), jnp.float32)
            accn = jnp.zeros((16,), jnp.float32)
            for d in range(EMB):
                col = jnp.full((16,), d, jnp.int32)
                ug = plsc.load_gather(u_b, [rows, zero, col])
                pg = plsc.load_gather(p_b, [rows, zero, col])
                ng = plsc.load_gather(n_b, [rows, zero, col])
                accp = accp + ug * pg
                accn = accn + ug * ng
            base = j * CHUNK + blk * 16
            op_v[pl.ds(base, 16)] = accp
            on_v[pl.ds(base, 16)] = accn
            return carry

        lax.fori_loop(0, CHUNK // 16, block, 0)

    fire_chunk(0)
    for j in range(NCHUNK):
        if j + 1 < NCHUNK:
            fire_chunk(j + 1)
        wait_chunk(j)
        compute_chunk(j)

    out = pl.ds(wid * BPW, BPW)
    pltpu.sync_copy(op_v, out_pos_h.at[out])
    pltpu.sync_copy(on_v, out_neg_h.at[out])


@jax.jit
def kernel(user_ids, pos_ids, neg_ids, user_emb, movie_emb):
    uids = user_ids.astype(jnp.int32)
    pids = pos_ids.astype(jnp.int32)
    nids = neg_ids.astype(jnp.int32)

    mesh = plsc.VectorSubcoreMesh(
        core_axis_name="c", subcore_axis_name="s", num_cores=2, num_subcores=16
    )
    buf = pltpu.VMEM((CHUNK, 1, EMB), jnp.float32)
    run = pl.kernel(
        _sc_kernel,
        out_type=(
            jax.ShapeDtypeStruct((B,), jnp.float32),
            jax.ShapeDtypeStruct((B,), jnp.float32),
        ),
        mesh=mesh,
        scratch_types=[
            pltpu.VMEM((BPW,), jnp.int32),
            pltpu.VMEM((BPW,), jnp.int32),
            pltpu.VMEM((BPW,), jnp.int32),
            buf, buf, buf, buf, buf, buf,
            pltpu.VMEM((BPW,), jnp.float32),
            pltpu.VMEM((BPW,), jnp.float32),
            pltpu.SemaphoreType.DMA,
            pltpu.SemaphoreType.DMA,
        ],
        compiler_params=pltpu.CompilerParams(needs_layout_passes=False),
    )
    return run(uids, pids, nids, user_emb, movie_emb)


# speedup vs baseline: 1.4819x; 1.0026x over previous
import jax
import jax.numpy as jnp
from jax import lax
from jax.experimental import pallas as pl
from jax.experimental.pallas import tpu as pltpu
from jax.experimental.pallas import tpu_sc as plsc

B = 16384
EMB = 32
NW = 32
BPW = B // NW
CHUNK = 128
NCHUNK = BPW // CHUNK


def _sc_kernel(user_ids_h, pos_ids_h, neg_ids_h, user_emb_h, movie_emb_h,
               out_pos_h, out_neg_h,
               uidx_v, pidx_v, nidx_v,
               u_b0, u_b1, p_b0, p_b1, n_b0, n_b1,
               op_v, on_v, *sems_flat):
    wid = lax.axis_index("s") * 2 + lax.axis_index("c")

    pltpu.sync_copy(user_ids_h.at[pl.ds(wid * BPW, BPW)], uidx_v)
    pltpu.sync_copy(pos_ids_h.at[pl.ds(wid * BPW, BPW)], pidx_v)
    pltpu.sync_copy(neg_ids_h.at[pl.ds(wid * BPW, BPW)], nidx_v)

    bufs = ((u_b0, p_b0, n_b0), (u_b1, p_b1, n_b1))
    NSEM = 8
    sems = (sems_flat[:NSEM], sems_flat[NSEM:])

    def fire_chunk(j):
        u_b, p_b, n_b = bufs[j % 2]
        psems = sems[j % 2]

        def grp(g, carry):
            src = pl.ds(j * CHUNK + g * 16, 16)
            uids16 = uidx_v[src]
            pids16 = pidx_v[src]
            nids16 = nidx_v[src]
            for k in range(16):
                dst = g * 16 + k
                pltpu.async_copy(
                    user_emb_h.at[pl.ds(uids16[k], 1)], u_b.at[dst],
                    psems[(3 * k) % NSEM])
                pltpu.async_copy(
                    movie_emb_h.at[pl.ds(pids16[k], 1)], p_b.at[dst],
                    psems[(3 * k + 1) % NSEM])
                pltpu.async_copy(
                    movie_emb_h.at[pl.ds(nids16[k], 1)], n_b.at[dst],
                    psems[(3 * k + 2) % NSEM])
            return carry

        lax.fori_loop(0, CHUNK // 16, grp, 0)

    def wait_chunk(j):
        psems = sems[j % 2]
        per_sem = 3 * CHUNK // NSEM
        for s in range(NSEM):
            pltpu.make_async_copy(
                user_emb_h.at[pl.ds(0, per_sem)],
                bufs[j % 2][0].at[pl.ds(0, per_sem), 0],
                psems[s],
            ).wait()

    def compute_chunk(j):
        u_b, p_b, n_b = bufs[j % 2]

        def block(blk, carry):
            rows = blk * 16 + lax.iota(jnp.int32, 16)
            zero = jnp.zeros((16,), jnp.int32)
            accp = jnp.zeros((16,), jnp.float32)
            accn = jnp.zeros((16,), jnp.float32)
            for d in range(EMB):
                col = jnp.full((16,), d, jnp.int32)
                ug = plsc.load_gather(u_b, [rows, zero, col])
                pg = plsc.load_gather(p_b, [rows, zero, col])
                ng = plsc.load_gather(n_b, [rows, zero, col])
                accp = accp + ug * pg
                accn = accn + ug * ng
            base = j * CHUNK + blk * 16
            op_v[pl.ds(base, 16)] = accp
            on_v[pl.ds(base, 16)] = accn
            return carry

        lax.fori_loop(0, CHUNK // 16, block, 0)

    fire_chunk(0)
    for j in range(NCHUNK):
        if j + 1 < NCHUNK:
            fire_chunk(j + 1)
        wait_chunk(j)
        compute_chunk(j)

    out = pl.ds(wid * BPW, BPW)
    pltpu.sync_copy(op_v, out_pos_h.at[out])
    pltpu.sync_copy(on_v, out_neg_h.at[out])


@jax.jit
def kernel(user_ids, pos_ids, neg_ids, user_emb, movie_emb):
    uids = user_ids.astype(jnp.int32)
    pids = pos_ids.astype(jnp.int32)
    nids = neg_ids.astype(jnp.int32)

    mesh = plsc.VectorSubcoreMesh(
        core_axis_name="c", subcore_axis_name="s", num_cores=2, num_subcores=16
    )
    buf = pltpu.VMEM((CHUNK, 1, EMB), jnp.float32)
    run = pl.kernel(
        _sc_kernel,
        out_type=(
            jax.ShapeDtypeStruct((B,), jnp.float32),
            jax.ShapeDtypeStruct((B,), jnp.float32),
        ),
        mesh=mesh,
        scratch_types=[
            pltpu.VMEM((BPW,), jnp.int32),
            pltpu.VMEM((BPW,), jnp.int32),
            pltpu.VMEM((BPW,), jnp.int32),
            buf, buf, buf, buf, buf, buf,
            pltpu.VMEM((BPW,), jnp.float32),
            pltpu.VMEM((BPW,), jnp.float32),
        ] + [pltpu.SemaphoreType.DMA] * 16,
        compiler_params=pltpu.CompilerParams(needs_layout_passes=False),
    )
    return run(uids, pids, nids, user_emb, movie_emb)
